# baseline (speedup 1.0000x reference)
import functools

import jax
import jax.numpy as jnp
from jax import lax
from jax.experimental import pallas as pl
from jax.experimental.pallas import tpu as pltpu
from jax.experimental.pallas import tpu_sc as plsc

_N = 10000
_E = 320000
_D_IN = 128
_D_HID = 128
_D_OUT = 256

_NC = 2
_NS = 16
_NH = _N // _NC
_ACC_R = _NH + 8
_K = 40
_EPW = _E // _NS
_CPW = _EPW // _K
_NBUF = 2
_IDXS = 4
_RPT = 312
_TAIL = _ACC_R - _NS * _RPT

_DGR = 40


def _make_sc_agg(with_deg):
    mesh = plsc.VectorSubcoreMesh(
        core_axis_name="c", subcore_axis_name="s",
        num_cores=_NC, num_subcores=_NS)

    out_type = [jax.ShapeDtypeStruct((_NC, _ACC_R, _D_IN), jnp.float32)]
    scratch = [
        pltpu.VMEM((_IDXS, 4, _K), jnp.int32),
        pltpu.VMEM((_NBUF, _K, _D_IN), jnp.float32),
        pltpu.VMEM_SHARED((_ACC_R, _D_IN), jnp.float32),
        pltpu.SemaphoreType.DMA((_NBUF,)),
        pltpu.SemaphoreType.DMA((_IDXS,)),
    ]
    if with_deg:
        out_type.append(jax.ShapeDtypeStruct((_NC, _DGR, 128), jnp.float32))
        scratch += [
            pltpu.VMEM((_K, 128), jnp.float32),
            pltpu.VMEM_SHARED((_DGR, 128), jnp.float32),
        ]

    @functools.partial(
        pl.kernel, out_type=tuple(out_type), mesh=mesh,
        scratch_types=scratch)
    def agg(*refs):
        if with_deg:
            (x_hbm, tbl_hbm, zero_hbm, ident_hbm,
             out_hbm, deg_hbm,
             idx_v, rows_v, acc, sems_g, sems_i, orow_v, deg_acc) = refs
        else:
            (x_hbm, tbl_hbm, zero_hbm,
             out_hbm,
             idx_v, rows_v, acc, sems_g, sems_i) = refs
        c = lax.axis_index("c")
        s = lax.axis_index("s")

        pltpu.sync_copy(zero_hbm.at[pl.ds(s * _RPT, _RPT)],
                        acc.at[pl.ds(s * _RPT, _RPT)])

        @pl.when(s == _NS - 1)
        def _zero_tail():
            pltpu.sync_copy(zero_hbm.at[pl.ds(_NS * _RPT, _TAIL)],
                            acc.at[pl.ds(_NS * _RPT, _TAIL)])

        if with_deg:
            @pl.when(s < _DGR // 8)
            def _zero_deg():
                pltpu.sync_copy(zero_hbm.at[pl.ds(s * 8, 8)],
                                deg_acc.at[pl.ds(s * 8, 8)])

        plsc.subcore_barrier()

        def i_copy(g, t):
            return pltpu.make_async_copy(
                tbl_hbm.at[c, s, g], idx_v.at[t], sems_i.at[t])

        def g_copy(g, b):
            return pltpu.make_async_copy(
                x_hbm.at[idx_v.at[lax.rem(g, _IDXS), 0]], rows_v.at[b],
                sems_g.at[b])

        def chunk_body(g, b, last):
            sl = lax.rem(g, _IDXS)
            if with_deg:
                pltpu.sync_copy(ident_hbm.at[idx_v.at[sl, 3]], orow_v)
                pltpu.sync_copy(orow_v, deg_acc.at[idx_v.at[sl, 2]],
                                add=True)
            g_copy(g, b).wait()
            pltpu.sync_copy(rows_v.at[b], acc.at[idx_v.at[sl, 1]], add=True)
            if not last:
                i_copy(g + _IDXS, sl).start()
                i_copy(g + _NBUF, lax.rem(g + _NBUF, _IDXS)).wait()
                g_copy(g + _NBUF, b).start()

        for t in range(_IDXS):
            i_copy(t, t).start()
        for b in range(_NBUF):
            i_copy(b, b).wait()
            g_copy(b, b).start()

        def super_step(ss, carry):
            for b in range(_NBUF):
                chunk_body(ss * _NBUF + b, b, last=False)
            return carry

        lax.fori_loop(0, _CPW // _NBUF - 1, super_step, 0)
        for b in range(_NBUF):
            chunk_body(_CPW - _NBUF + b, b, last=True)
        for b in range(_NBUF):
            g = _CPW + b
            i_copy(g, lax.rem(g, _IDXS)).wait()

        plsc.subcore_barrier()
        pltpu.sync_copy(acc.at[pl.ds(s * _RPT, _RPT)],
                        out_hbm.at[c, pl.ds(s * _RPT, _RPT)])

        @pl.when(s == _NS - 1)
        def _write_tail():
            pltpu.sync_copy(acc.at[pl.ds(_NS * _RPT, _TAIL)],
                            out_hbm.at[c, pl.ds(_NS * _RPT, _TAIL)])

        if with_deg:
            @pl.when(s < _DGR // 8)
            def _write_deg():
                pltpu.sync_copy(deg_acc.at[pl.ds(s * 8, 8)],
                                deg_hbm.at[c, pl.ds(s * 8, 8)])

    return agg


@functools.cache
def _sc_agg(with_deg):
    return _make_sc_agg(with_deg)


_BLK = 2000


def _tc1_body(p_ref, deg_ref, x_ref, wl_ref, b_ref, wr_ref, h_ref, di_ref):
    deg = jnp.maximum(deg_ref[...], 1.0)
    deg_inv = 1.0 / deg
    agg = p_ref[...] * deg_inv
    h = (jnp.dot(agg, wl_ref[...], preferred_element_type=jnp.float32,
                 precision=lax.Precision.HIGHEST)
         + b_ref[...]
         + jnp.dot(x_ref[...], wr_ref[...], preferred_element_type=jnp.float32,
                   precision=lax.Precision.HIGHEST))
    h = jax.nn.sigmoid(h)
    nrm = jnp.sqrt(jnp.sum(h * h, axis=1, keepdims=True))
    h_ref[...] = h / jnp.maximum(nrm, 1e-12)
    di_ref[...] = deg_inv


def _tc1(agg, deg_col, x, W1_l, b1, W1_r):
    grid = _N // _BLK
    return pl.pallas_call(
        _tc1_body,
        grid=(grid,),
        in_specs=[
            pl.BlockSpec((_BLK, _D_IN), lambda i: (i, 0)),
            pl.BlockSpec((_BLK, 1), lambda i: (i, 0)),
            pl.BlockSpec((_BLK, _D_IN), lambda i: (i, 0)),
            pl.BlockSpec((_D_IN, _D_HID), lambda i: (0, 0)),
            pl.BlockSpec((1, _D_HID), lambda i: (0, 0)),
            pl.BlockSpec((_D_IN, _D_HID), lambda i: (0, 0)),
        ],
        out_specs=[
            pl.BlockSpec((_BLK, _D_HID), lambda i: (i, 0)),
            pl.BlockSpec((_BLK, 1), lambda i: (i, 0)),
        ],
        out_shape=[
            jax.ShapeDtypeStruct((_N, _D_HID), jnp.float32),
            jax.ShapeDtypeStruct((_N, 1), jnp.float32),
        ],
    )(agg, deg_col, x, W1_l, b1.reshape(1, -1), W1_r)


def _tc2_body(p_ref, di_ref, h_ref, wl_ref, b_ref, wr_ref, o_ref):
    agg = p_ref[...] * di_ref[...]
    h = (jnp.dot(agg, wl_ref[...], preferred_element_type=jnp.float32,
                 precision=lax.Precision.HIGHEST)
         + b_ref[...]
         + jnp.dot(h_ref[...], wr_ref[...], preferred_element_type=jnp.float32,
                   precision=lax.Precision.HIGHEST))
    h = jax.nn.sigmoid(h)
    nrm = jnp.sqrt(jnp.sum(h * h, axis=1, keepdims=True))
    h = h / jnp.maximum(nrm, 1e-12)
    m = jnp.max(h, axis=1, keepdims=True)
    lse = m + jnp.log(jnp.sum(jnp.exp(h - m), axis=1, keepdims=True))
    o_ref[...] = h - lse


def _tc2(agg, deg_inv, h1, W2_l, b2, W2_r):
    grid = _N // _BLK
    return pl.pallas_call(
        _tc2_body,
        grid=(grid,),
        in_specs=[
            pl.BlockSpec((_BLK, _D_HID), lambda i: (i, 0)),
            pl.BlockSpec((_BLK, 1), lambda i: (i, 0)),
            pl.BlockSpec((_BLK, _D_HID), lambda i: (i, 0)),
            pl.BlockSpec((_D_HID, _D_OUT), lambda i: (0, 0)),
            pl.BlockSpec((1, _D_OUT), lambda i: (0, 0)),
            pl.BlockSpec((_D_HID, _D_OUT), lambda i: (0, 0)),
        ],
        out_specs=pl.BlockSpec((_BLK, _D_OUT), lambda i: (i, 0)),
        out_shape=jax.ShapeDtypeStruct((_N, _D_OUT), jnp.float32),
    )(agg, deg_inv, h1, W2_l, b2.reshape(1, -1), W2_r)


def _edge_tables(edge_index):
    srcf = edge_index[0]
    dstf = edge_index[1]
    rows = []
    for cc in range(_NC):
        ln = dstf - cc * _NH
        inc = (dstf >= cc * _NH) & (dstf < (cc + 1) * _NH)
        dstc = jnp.where(inc, ln, _NH)
        rows.append(jnp.stack([
            srcf,
            dstc,
            jax.lax.shift_right_logical(dstc, 7),
            jax.lax.bitwise_and(dstc, 127),
        ]))
    tbl = jnp.stack(rows)
    tbl = tbl.reshape(_NC, 4, _NS, _CPW, _K).transpose(0, 2, 3, 1, 4)
    pad = jnp.zeros((_NC, _NS, _IDXS, 4, _K), jnp.int32)
    return jnp.concatenate([tbl, pad], axis=2)


def kernel(x, edge_index, W1_l, b1, W1_r, W2_l, b2, W2_r):
    tbl = _edge_tables(edge_index)
    ident = jnp.eye(128, dtype=jnp.float32)
    zeros = jnp.zeros((_ACC_R, _D_IN), jnp.float32)

    parts1, deg = _sc_agg(True)(x, tbl, zeros, ident)
    agg1 = parts1[:, :_NH].reshape(_N, _D_IN)
    deg_col = deg.reshape(_NC, _DGR * 128)[:, :_NH].reshape(_N, 1)
    h1, deg_inv = _tc1(agg1, deg_col, x, W1_l, b1, W1_r)
    parts2 = _sc_agg(False)(h1, tbl, zeros)[0]
    agg2 = parts2[:, :_NH].reshape(_N, _D_HID)
    return _tc2(agg2, deg_inv, h1, W2_l, b2, W2_r)


# speedup vs baseline: 1.0008x; 1.0008x over previous
import functools

import jax
import jax.numpy as jnp
from jax import lax
from jax.experimental import pallas as pl
from jax.experimental.pallas import tpu as pltpu
from jax.experimental.pallas import tpu_sc as plsc

_N = 10000
_E = 320000
_D_IN = 128
_D_HID = 128
_D_OUT = 256

_NC = 2
_NS = 16
_NH = _N // _NC
_ACC_R = _NH + 8
_K = 40
_EPW = _E // _NS
_CPW = _EPW // _K
_NBUF = 2
_IDXS = 4
_RPT = 312
_TAIL = _ACC_R - _NS * _RPT

_DGR = 40


def _make_sc_agg(with_deg):
    mesh = plsc.VectorSubcoreMesh(
        core_axis_name="c", subcore_axis_name="s",
        num_cores=_NC, num_subcores=_NS)

    out_type = [jax.ShapeDtypeStruct((_NC, _ACC_R, _D_IN), jnp.float32)]
    scratch = [
        pltpu.VMEM((_IDXS, 4, _K), jnp.int32),
        pltpu.VMEM((_NBUF, _K, _D_IN), jnp.float32),
        pltpu.VMEM_SHARED((_ACC_R, _D_IN), jnp.float32),
        pltpu.SemaphoreType.DMA((_NBUF,)),
        pltpu.SemaphoreType.DMA((_IDXS,)),
    ]
    if with_deg:
        out_type.append(jax.ShapeDtypeStruct((_NC, _DGR, 128), jnp.float32))
        scratch += [
            pltpu.VMEM((_NBUF, _K, 128), jnp.float32),
            pltpu.VMEM_SHARED((_DGR, 128), jnp.float32),
            pltpu.SemaphoreType.DMA((_NBUF,)),
        ]

    @functools.partial(
        pl.kernel, out_type=tuple(out_type), mesh=mesh,
        scratch_types=scratch)
    def agg(*refs):
        if with_deg:
            (x_hbm, tbl_hbm, zero_hbm, ident_hbm,
             out_hbm, deg_hbm,
             idx_v, rows_v, acc, sems_g, sems_i,
             orow_v, deg_acc, sems_o) = refs
        else:
            (x_hbm, tbl_hbm, zero_hbm,
             out_hbm,
             idx_v, rows_v, acc, sems_g, sems_i) = refs
        c = lax.axis_index("c")
        s = lax.axis_index("s")

        pltpu.sync_copy(zero_hbm.at[pl.ds(s * _RPT, _RPT)],
                        acc.at[pl.ds(s * _RPT, _RPT)])

        @pl.when(s == _NS - 1)
        def _zero_tail():
            pltpu.sync_copy(zero_hbm.at[pl.ds(_NS * _RPT, _TAIL)],
                            acc.at[pl.ds(_NS * _RPT, _TAIL)])

        if with_deg:
            @pl.when(s < _DGR // 8)
            def _zero_deg():
                pltpu.sync_copy(zero_hbm.at[pl.ds(s * 8, 8)],
                                deg_acc.at[pl.ds(s * 8, 8)])

        plsc.subcore_barrier()

        def i_copy(g, t):
            return pltpu.make_async_copy(
                tbl_hbm.at[c, s, g], idx_v.at[t], sems_i.at[t])

        def g_copy(g, b):
            return pltpu.make_async_copy(
                x_hbm.at[idx_v.at[lax.rem(g, _IDXS), 0]], rows_v.at[b],
                sems_g.at[b])

        def o_copy(g, b):
            return pltpu.make_async_copy(
                ident_hbm.at[idx_v.at[lax.rem(g, _IDXS), 3]], orow_v.at[b],
                sems_o.at[b])

        def chunk_body(g, b, last):
            sl = lax.rem(g, _IDXS)
            if with_deg:
                o_copy(g, b).wait()
                pltpu.sync_copy(orow_v.at[b], deg_acc.at[idx_v.at[sl, 2]],
                                add=True)
            g_copy(g, b).wait()
            pltpu.sync_copy(rows_v.at[b], acc.at[idx_v.at[sl, 1]], add=True)
            if not last:
                i_copy(g + _IDXS, sl).start()
                i_copy(g + _NBUF, lax.rem(g + _NBUF, _IDXS)).wait()
                g_copy(g + _NBUF, b).start()
                if with_deg:
                    o_copy(g + _NBUF, b).start()

        for t in range(_IDXS):
            i_copy(t, t).start()
        for b in range(_NBUF):
            i_copy(b, b).wait()
            g_copy(b, b).start()
            if with_deg:
                o_copy(b, b).start()

        def super_step(ss, carry):
            for b in range(_NBUF):
                chunk_body(ss * _NBUF + b, b, last=False)
            return carry

        lax.fori_loop(0, _CPW // _NBUF - 1, super_step, 0)
        for b in range(_NBUF):
            chunk_body(_CPW - _NBUF + b, b, last=True)
        for b in range(_NBUF):
            g = _CPW + b
            i_copy(g, lax.rem(g, _IDXS)).wait()

        plsc.subcore_barrier()
        pltpu.sync_copy(acc.at[pl.ds(s * _RPT, _RPT)],
                        out_hbm.at[c, pl.ds(s * _RPT, _RPT)])

        @pl.when(s == _NS - 1)
        def _write_tail():
            pltpu.sync_copy(acc.at[pl.ds(_NS * _RPT, _TAIL)],
                            out_hbm.at[c, pl.ds(_NS * _RPT, _TAIL)])

        if with_deg:
            @pl.when(s < _DGR // 8)
            def _write_deg():
                pltpu.sync_copy(deg_acc.at[pl.ds(s * 8, 8)],
                                deg_hbm.at[c, pl.ds(s * 8, 8)])

    return agg


@functools.cache
def _sc_agg(with_deg):
    return _make_sc_agg(with_deg)


_BLK = 2000


def _tc1_body(p_ref, deg_ref, x_ref, wl_ref, b_ref, wr_ref, h_ref, di_ref):
    deg = jnp.maximum(deg_ref[...], 1.0)
    deg_inv = 1.0 / deg
    agg = p_ref[...] * deg_inv
    h = (jnp.dot(agg, wl_ref[...], preferred_element_type=jnp.float32,
                 precision=lax.Precision.HIGHEST)
         + b_ref[...]
         + jnp.dot(x_ref[...], wr_ref[...], preferred_element_type=jnp.float32,
                   precision=lax.Precision.HIGHEST))
    h = jax.nn.sigmoid(h)
    nrm = jnp.sqrt(jnp.sum(h * h, axis=1, keepdims=True))
    h_ref[...] = h / jnp.maximum(nrm, 1e-12)
    di_ref[...] = deg_inv


def _tc1(agg, deg_col, x, W1_l, b1, W1_r):
    grid = _N // _BLK
    return pl.pallas_call(
        _tc1_body,
        grid=(grid,),
        in_specs=[
            pl.BlockSpec((_BLK, _D_IN), lambda i: (i, 0)),
            pl.BlockSpec((_BLK, 1), lambda i: (i, 0)),
            pl.BlockSpec((_BLK, _D_IN), lambda i: (i, 0)),
            pl.BlockSpec((_D_IN, _D_HID), lambda i: (0, 0)),
            pl.BlockSpec((1, _D_HID), lambda i: (0, 0)),
            pl.BlockSpec((_D_IN, _D_HID), lambda i: (0, 0)),
        ],
        out_specs=[
            pl.BlockSpec((_BLK, _D_HID), lambda i: (i, 0)),
            pl.BlockSpec((_BLK, 1), lambda i: (i, 0)),
        ],
        out_shape=[
            jax.ShapeDtypeStruct((_N, _D_HID), jnp.float32),
            jax.ShapeDtypeStruct((_N, 1), jnp.float32),
        ],
    )(agg, deg_col, x, W1_l, b1.reshape(1, -1), W1_r)


def _tc2_body(p_ref, di_ref, h_ref, wl_ref, b_ref, wr_ref, o_ref):
    agg = p_ref[...] * di_ref[...]
    h = (jnp.dot(agg, wl_ref[...], preferred_element_type=jnp.float32,
                 precision=lax.Precision.HIGHEST)
         + b_ref[...]
         + jnp.dot(h_ref[...], wr_ref[...], preferred_element_type=jnp.float32,
                   precision=lax.Precision.HIGHEST))
    h = jax.nn.sigmoid(h)
    nrm = jnp.sqrt(jnp.sum(h * h, axis=1, keepdims=True))
    h = h / jnp.maximum(nrm, 1e-12)
    m = jnp.max(h, axis=1, keepdims=True)
    lse = m + jnp.log(jnp.sum(jnp.exp(h - m), axis=1, keepdims=True))
    o_ref[...] = h - lse


def _tc2(agg, deg_inv, h1, W2_l, b2, W2_r):
    grid = _N // _BLK
    return pl.pallas_call(
        _tc2_body,
        grid=(grid,),
        in_specs=[
            pl.BlockSpec((_BLK, _D_HID), lambda i: (i, 0)),
            pl.BlockSpec((_BLK, 1), lambda i: (i, 0)),
            pl.BlockSpec((_BLK, _D_HID), lambda i: (i, 0)),
            pl.BlockSpec((_D_HID, _D_OUT), lambda i: (0, 0)),
            pl.BlockSpec((1, _D_OUT), lambda i: (0, 0)),
            pl.BlockSpec((_D_HID, _D_OUT), lambda i: (0, 0)),
        ],
        out_specs=pl.BlockSpec((_BLK, _D_OUT), lambda i: (i, 0)),
        out_shape=jax.ShapeDtypeStruct((_N, _D_OUT), jnp.float32),
    )(agg, deg_inv, h1, W2_l, b2.reshape(1, -1), W2_r)


def _edge_tables(edge_index):
    srcf = edge_index[0]
    dstf = edge_index[1]
    rows = []
    for cc in range(_NC):
        ln = dstf - cc * _NH
        inc = (dstf >= cc * _NH) & (dstf < (cc + 1) * _NH)
        dstc = jnp.where(inc, ln, _NH)
        rows.append(jnp.stack([
            srcf,
            dstc,
            jax.lax.shift_right_logical(dstc, 7),
            jax.lax.bitwise_and(dstc, 127),
        ]))
    tbl = jnp.stack(rows)
    tbl = tbl.reshape(_NC, 4, _NS, _CPW, _K).transpose(0, 2, 3, 1, 4)
    pad = jnp.zeros((_NC, _NS, _IDXS, 4, _K), jnp.int32)
    return jnp.concatenate([tbl, pad], axis=2)


def kernel(x, edge_index, W1_l, b1, W1_r, W2_l, b2, W2_r):
    tbl = _edge_tables(edge_index)
    ident = jnp.eye(128, dtype=jnp.float32)
    zeros = jnp.zeros((_ACC_R, _D_IN), jnp.float32)

    parts1, deg = _sc_agg(True)(x, tbl, zeros, ident)
    agg1 = parts1[:, :_NH].reshape(_N, _D_IN)
    deg_col = deg.reshape(_NC, _DGR * 128)[:, :_NH].reshape(_N, 1)
    h1, deg_inv = _tc1(agg1, deg_col, x, W1_l, b1, W1_r)
    parts2 = _sc_agg(False)(h1, tbl, zeros)[0]
    agg2 = parts2[:, :_NH].reshape(_N, _D_HID)
    return _tc2(agg2, deg_inv, h1, W2_l, b2, W2_r)


# speedup vs baseline: 4.9579x; 4.9538x over previous
import functools

import jax
import jax.numpy as jnp
from jax import lax
from jax.experimental import pallas as pl
from jax.experimental.pallas import tpu as pltpu
from jax.experimental.pallas import tpu_sc as plsc

_N = 10000
_E = 320000
_D_IN = 128
_D_HID = 128
_D_OUT = 256

_NC = 2
_NS = 16
_NH = _N // _NC
_ACC_R = _NH + 8
_K = 40
_EPW = _E // _NS
_CPW = _EPW // _K
_NBUF = 2
_IDXS = 4
_RPT = 312
_TAIL = _ACC_R - _NS * _RPT

_DGR = 40


def _make_sc_agg(with_deg):
    mesh = plsc.VectorSubcoreMesh(
        core_axis_name="c", subcore_axis_name="s",
        num_cores=_NC, num_subcores=_NS)

    out_type = [jax.ShapeDtypeStruct((_NC, _ACC_R, _D_IN), jnp.float32)]
    scratch = [
        pltpu.VMEM((_IDXS, 4, _K), jnp.int32),
        pltpu.VMEM((_NBUF, _K, _D_IN), jnp.float32),
        pltpu.VMEM_SHARED((_ACC_R, _D_IN), jnp.float32),
        pltpu.SemaphoreType.DMA((_NBUF,)),
        pltpu.SemaphoreType.DMA((_IDXS,)),
    ]
    if with_deg:
        out_type.append(
            jax.ShapeDtypeStruct((_NC, _NS * _DGR, 128), jnp.float32))
        scratch += [
            pltpu.VMEM((_NBUF, _K, 128), jnp.float32),
            pltpu.VMEM_SHARED((_NS * _DGR, 128), jnp.float32),
            pltpu.SemaphoreType.DMA((_NBUF,)),
        ]

    @functools.partial(
        pl.kernel, out_type=tuple(out_type), mesh=mesh,
        scratch_types=scratch)
    def agg(*refs):
        if with_deg:
            (x_hbm, tbl_hbm, zero_hbm, ident_hbm,
             out_hbm, deg_hbm,
             idx_v, rows_v, acc, sems_g, sems_i,
             orow_v, deg_acc, sems_o) = refs
        else:
            (x_hbm, tbl_hbm, zero_hbm,
             out_hbm,
             idx_v, rows_v, acc, sems_g, sems_i) = refs
        c = lax.axis_index("c")
        s = lax.axis_index("s")

        pltpu.sync_copy(zero_hbm.at[pl.ds(s * _RPT, _RPT)],
                        acc.at[pl.ds(s * _RPT, _RPT)])

        @pl.when(s == _NS - 1)
        def _zero_tail():
            pltpu.sync_copy(zero_hbm.at[pl.ds(_NS * _RPT, _TAIL)],
                            acc.at[pl.ds(_NS * _RPT, _TAIL)])

        if with_deg:
            pltpu.sync_copy(zero_hbm.at[pl.ds(0, _DGR)],
                            deg_acc.at[pl.ds(s * _DGR, _DGR)])

        plsc.subcore_barrier()

        def i_copy(g, t):
            return pltpu.make_async_copy(
                tbl_hbm.at[c, s, g], idx_v.at[t], sems_i.at[t])

        def g_copy(g, b):
            return pltpu.make_async_copy(
                x_hbm.at[idx_v.at[lax.rem(g, _IDXS), 0]], rows_v.at[b],
                sems_g.at[b])

        def o_copy(g, b):
            return pltpu.make_async_copy(
                ident_hbm.at[idx_v.at[lax.rem(g, _IDXS), 3]], orow_v.at[b],
                sems_o.at[b])

        def chunk_body(g, b, last):
            sl = lax.rem(g, _IDXS)
            if with_deg:
                o_copy(g, b).wait()
                pltpu.sync_copy(orow_v.at[b], deg_acc.at[idx_v.at[sl, 2]],
                                add=True)
            g_copy(g, b).wait()
            pltpu.sync_copy(rows_v.at[b], acc.at[idx_v.at[sl, 1]], add=True)
            if not last:
                i_copy(g + _IDXS, sl).start()
                i_copy(g + _NBUF, lax.rem(g + _NBUF, _IDXS)).wait()
                g_copy(g + _NBUF, b).start()
                if with_deg:
                    o_copy(g + _NBUF, b).start()

        for t in range(_IDXS):
            i_copy(t, t).start()
        for b in range(_NBUF):
            i_copy(b, b).wait()
            g_copy(b, b).start()
            if with_deg:
                o_copy(b, b).start()

        def super_step(ss, carry):
            for b in range(_NBUF):
                chunk_body(ss * _NBUF + b, b, last=False)
            return carry

        lax.fori_loop(0, _CPW // _NBUF - 1, super_step, 0)
        for b in range(_NBUF):
            chunk_body(_CPW - _NBUF + b, b, last=True)
        for b in range(_NBUF):
            g = _CPW + b
            i_copy(g, lax.rem(g, _IDXS)).wait()

        plsc.subcore_barrier()
        pltpu.sync_copy(acc.at[pl.ds(s * _RPT, _RPT)],
                        out_hbm.at[c, pl.ds(s * _RPT, _RPT)])

        @pl.when(s == _NS - 1)
        def _write_tail():
            pltpu.sync_copy(acc.at[pl.ds(_NS * _RPT, _TAIL)],
                            out_hbm.at[c, pl.ds(_NS * _RPT, _TAIL)])

        if with_deg:
            pltpu.sync_copy(deg_acc.at[pl.ds(s * _DGR, _DGR)],
                            deg_hbm.at[c, pl.ds(s * _DGR, _DGR)])

    return agg


@functools.cache
def _sc_agg(with_deg):
    return _make_sc_agg(with_deg)


_BLK = 2000


def _tc1_body(p_ref, deg_ref, x_ref, wl_ref, b_ref, wr_ref, h_ref, di_ref):
    deg = jnp.maximum(deg_ref[...], 1.0)
    deg_inv = 1.0 / deg
    agg = p_ref[...] * deg_inv
    h = (jnp.dot(agg, wl_ref[...], preferred_element_type=jnp.float32,
                 precision=lax.Precision.HIGHEST)
         + b_ref[...]
         + jnp.dot(x_ref[...], wr_ref[...], preferred_element_type=jnp.float32,
                   precision=lax.Precision.HIGHEST))
    h = jax.nn.sigmoid(h)
    nrm = jnp.sqrt(jnp.sum(h * h, axis=1, keepdims=True))
    h_ref[...] = h / jnp.maximum(nrm, 1e-12)
    di_ref[...] = deg_inv


def _tc1(agg, deg_col, x, W1_l, b1, W1_r):
    grid = _N // _BLK
    return pl.pallas_call(
        _tc1_body,
        grid=(grid,),
        in_specs=[
            pl.BlockSpec((_BLK, _D_IN), lambda i: (i, 0)),
            pl.BlockSpec((_BLK, 1), lambda i: (i, 0)),
            pl.BlockSpec((_BLK, _D_IN), lambda i: (i, 0)),
            pl.BlockSpec((_D_IN, _D_HID), lambda i: (0, 0)),
            pl.BlockSpec((1, _D_HID), lambda i: (0, 0)),
            pl.BlockSpec((_D_IN, _D_HID), lambda i: (0, 0)),
        ],
        out_specs=[
            pl.BlockSpec((_BLK, _D_HID), lambda i: (i, 0)),
            pl.BlockSpec((_BLK, 1), lambda i: (i, 0)),
        ],
        out_shape=[
            jax.ShapeDtypeStruct((_N, _D_HID), jnp.float32),
            jax.ShapeDtypeStruct((_N, 1), jnp.float32),
        ],
    )(agg, deg_col, x, W1_l, b1.reshape(1, -1), W1_r)


def _tc2_body(p_ref, di_ref, h_ref, wl_ref, b_ref, wr_ref, o_ref):
    agg = p_ref[...] * di_ref[...]
    h = (jnp.dot(agg, wl_ref[...], preferred_element_type=jnp.float32,
                 precision=lax.Precision.HIGHEST)
         + b_ref[...]
         + jnp.dot(h_ref[...], wr_ref[...], preferred_element_type=jnp.float32,
                   precision=lax.Precision.HIGHEST))
    h = jax.nn.sigmoid(h)
    nrm = jnp.sqrt(jnp.sum(h * h, axis=1, keepdims=True))
    h = h / jnp.maximum(nrm, 1e-12)
    m = jnp.max(h, axis=1, keepdims=True)
    lse = m + jnp.log(jnp.sum(jnp.exp(h - m), axis=1, keepdims=True))
    o_ref[...] = h - lse


def _tc2(agg, deg_inv, h1, W2_l, b2, W2_r):
    grid = _N // _BLK
    return pl.pallas_call(
        _tc2_body,
        grid=(grid,),
        in_specs=[
            pl.BlockSpec((_BLK, _D_HID), lambda i: (i, 0)),
            pl.BlockSpec((_BLK, 1), lambda i: (i, 0)),
            pl.BlockSpec((_BLK, _D_HID), lambda i: (i, 0)),
            pl.BlockSpec((_D_HID, _D_OUT), lambda i: (0, 0)),
            pl.BlockSpec((1, _D_OUT), lambda i: (0, 0)),
            pl.BlockSpec((_D_HID, _D_OUT), lambda i: (0, 0)),
        ],
        out_specs=pl.BlockSpec((_BLK, _D_OUT), lambda i: (i, 0)),
        out_shape=jax.ShapeDtypeStruct((_N, _D_OUT), jnp.float32),
    )(agg, deg_inv, h1, W2_l, b2.reshape(1, -1), W2_r)


def _edge_tables(edge_index):
    srcf = edge_index[0]
    dstf = edge_index[1]
    rows = []
    for cc in range(_NC):
        ln = dstf - cc * _NH
        inc = (dstf >= cc * _NH) & (dstf < (cc + 1) * _NH)
        dstc = jnp.where(inc, ln, _NH)
        rows.append(jnp.stack([
            srcf,
            dstc,
            jax.lax.shift_right_logical(dstc, 7),
            jax.lax.bitwise_and(dstc, 127),
        ]))
    tbl = jnp.stack(rows)
    tbl = tbl.reshape(_NC, 4, _NS, _CPW, _K)
    sid = jnp.arange(_NS, dtype=jnp.int32)[None, :, None, None]
    tbl = tbl.at[:, 2].add(sid * _DGR)
    tbl = tbl.at[:, 3].add(sid * 128)
    tbl = tbl.transpose(0, 2, 3, 1, 4)
    pad = jnp.zeros((_NC, _NS, _IDXS, 4, _K), jnp.int32)
    return jnp.concatenate([tbl, pad], axis=2)


def kernel(x, edge_index, W1_l, b1, W1_r, W2_l, b2, W2_r):
    tbl = _edge_tables(edge_index)
    ident = jnp.tile(jnp.eye(128, dtype=jnp.float32), (_NS, 1))
    zeros = jnp.zeros((_ACC_R, _D_IN), jnp.float32)

    parts1, deg = _sc_agg(True)(x, tbl, zeros, ident)
    agg1 = parts1[:, :_NH].reshape(_N, _D_IN)
    deg_col = (deg.reshape(_NC, _NS, _DGR * 128).sum(axis=1)[:, :_NH]
               .reshape(_N, 1))
    h1, deg_inv = _tc1(agg1, deg_col, x, W1_l, b1, W1_r)
    parts2 = _sc_agg(False)(h1, tbl, zeros)[0]
    agg2 = parts2[:, :_NH].reshape(_N, _D_HID)
    return _tc2(agg2, deg_inv, h1, W2_l, b2, W2_r)


# speedup vs baseline: 9.4639x; 1.9088x over previous
import functools

import jax
import jax.numpy as jnp
from jax import lax
from jax.experimental import pallas as pl
from jax.experimental.pallas import tpu as pltpu
from jax.experimental.pallas import tpu_sc as plsc

_N = 10000
_E = 320000
_D_IN = 128
_D_HID = 128
_D_OUT = 256

_NC = 2
_NS = 16
_NH = _N // _NC
_ACC_R = _NH + 8
_K = 40
_EPW = _E // _NS
_CPW = _EPW // _K
_NBUF = 2
_IDXS = 4
_RPT = 312
_TAIL = _ACC_R - _NS * _RPT

_DGR = 40


def _make_sc_agg(with_deg):
    mesh = plsc.VectorSubcoreMesh(
        core_axis_name="c", subcore_axis_name="s",
        num_cores=_NC, num_subcores=_NS)

    out_type = [jax.ShapeDtypeStruct((_NC, _ACC_R, _D_IN), jnp.float32)]
    scratch = [
        pltpu.VMEM((_IDXS, 4, _K), jnp.int32),
        pltpu.VMEM((_NBUF, _K, _D_IN), jnp.float32),
        pltpu.VMEM_SHARED((_ACC_R, _D_IN), jnp.float32),
        pltpu.SemaphoreType.DMA((_NBUF,)),
        pltpu.SemaphoreType.DMA((_IDXS,)),
    ]
    if with_deg:
        out_type.append(
            jax.ShapeDtypeStruct((_NC, _NS * _DGR, 128), jnp.float32))
        scratch += [
            pltpu.VMEM((_NBUF, _K, 128), jnp.float32),
            pltpu.VMEM_SHARED((_NS * _DGR, 128), jnp.float32),
            pltpu.SemaphoreType.DMA((_NBUF,)),
        ]

    @functools.partial(
        pl.kernel, out_type=tuple(out_type), mesh=mesh,
        scratch_types=scratch)
    def agg(*refs):
        if with_deg:
            (x_hbm, tbl_hbm, zero_hbm, ident_hbm,
             out_hbm, deg_hbm,
             idx_v, rows_v, acc, sems_g, sems_i,
             orow_v, deg_acc, sems_o) = refs
        else:
            (x_hbm, tbl_hbm, zero_hbm,
             out_hbm,
             idx_v, rows_v, acc, sems_g, sems_i) = refs
        c = lax.axis_index("c")
        s = lax.axis_index("s")

        pltpu.sync_copy(zero_hbm.at[pl.ds(s * _RPT, _RPT)],
                        acc.at[pl.ds(s * _RPT, _RPT)])

        @pl.when(s == _NS - 1)
        def _zero_tail():
            pltpu.sync_copy(zero_hbm.at[pl.ds(_NS * _RPT, _TAIL)],
                            acc.at[pl.ds(_NS * _RPT, _TAIL)])

        if with_deg:
            pltpu.sync_copy(zero_hbm.at[pl.ds(0, _DGR)],
                            deg_acc.at[pl.ds(s * _DGR, _DGR)])

        plsc.subcore_barrier()

        def i_copy(g, t):
            return pltpu.make_async_copy(
                tbl_hbm.at[c, s, g], idx_v.at[t], sems_i.at[t])

        def g_copy(g, b):
            return pltpu.make_async_copy(
                x_hbm.at[idx_v.at[lax.rem(g, _IDXS), 0]], rows_v.at[b],
                sems_g.at[b])

        def o_copy(g, b):
            return pltpu.make_async_copy(
                ident_hbm.at[idx_v.at[lax.rem(g, _IDXS), 3]], orow_v.at[b],
                sems_o.at[b])

        def chunk_body(g, b, last):
            sl = lax.rem(g, _IDXS)
            if with_deg:
                o_copy(g, b).wait()
                pltpu.sync_copy(orow_v.at[b], deg_acc.at[idx_v.at[sl, 2]],
                                add=True)
            g_copy(g, b).wait()
            pltpu.sync_copy(rows_v.at[b], acc.at[idx_v.at[sl, 1]], add=True)
            if not last:
                i_copy(g + _IDXS, sl).start()
                i_copy(g + _NBUF, lax.rem(g + _NBUF, _IDXS)).wait()
                g_copy(g + _NBUF, b).start()
                if with_deg:
                    o_copy(g + _NBUF, b).start()

        for t in range(_IDXS):
            i_copy(t, t).start()
        for b in range(_NBUF):
            i_copy(b, b).wait()
            g_copy(b, b).start()
            if with_deg:
                o_copy(b, b).start()

        def super_step(ss, carry):
            for b in range(_NBUF):
                chunk_body(ss * _NBUF + b, b, last=False)
            return carry

        lax.fori_loop(0, _CPW // _NBUF - 1, super_step, 0)
        for b in range(_NBUF):
            chunk_body(_CPW - _NBUF + b, b, last=True)
        for b in range(_NBUF):
            g = _CPW + b
            i_copy(g, lax.rem(g, _IDXS)).wait()

        plsc.subcore_barrier()
        pltpu.sync_copy(acc.at[pl.ds(s * _RPT, _RPT)],
                        out_hbm.at[c, pl.ds(s * _RPT, _RPT)])

        @pl.when(s == _NS - 1)
        def _write_tail():
            pltpu.sync_copy(acc.at[pl.ds(_NS * _RPT, _TAIL)],
                            out_hbm.at[c, pl.ds(_NS * _RPT, _TAIL)])

        if with_deg:
            pltpu.sync_copy(deg_acc.at[pl.ds(s * _DGR, _DGR)],
                            deg_hbm.at[c, pl.ds(s * _DGR, _DGR)])

    return agg


@functools.cache
def _sc_agg(with_deg):
    return _make_sc_agg(with_deg)


_BLK = 2000


def _tc1_body(p_ref, deg_ref, x_ref, wl_ref, b_ref, wr_ref, h_ref, di_ref):
    deg = jnp.maximum(deg_ref[...], 1.0)
    deg_inv = 1.0 / deg
    agg = p_ref[...] * deg_inv
    h = (jnp.dot(agg, wl_ref[...], preferred_element_type=jnp.float32,
                 precision=lax.Precision.HIGHEST)
         + b_ref[...]
         + jnp.dot(x_ref[...], wr_ref[...], preferred_element_type=jnp.float32,
                   precision=lax.Precision.HIGHEST))
    h = jax.nn.sigmoid(h)
    nrm = jnp.sqrt(jnp.sum(h * h, axis=1, keepdims=True))
    h_ref[...] = h / jnp.maximum(nrm, 1e-12)
    di_ref[...] = deg_inv


def _tc1(agg, deg_col, x, W1_l, b1, W1_r):
    grid = _N // _BLK
    return pl.pallas_call(
        _tc1_body,
        grid=(grid,),
        in_specs=[
            pl.BlockSpec((_BLK, _D_IN), lambda i: (i, 0)),
            pl.BlockSpec((_BLK, 1), lambda i: (i, 0)),
            pl.BlockSpec((_BLK, _D_IN), lambda i: (i, 0)),
            pl.BlockSpec((_D_IN, _D_HID), lambda i: (0, 0)),
            pl.BlockSpec((1, _D_HID), lambda i: (0, 0)),
            pl.BlockSpec((_D_IN, _D_HID), lambda i: (0, 0)),
        ],
        out_specs=[
            pl.BlockSpec((_BLK, _D_HID), lambda i: (i, 0)),
            pl.BlockSpec((_BLK, 1), lambda i: (i, 0)),
        ],
        out_shape=[
            jax.ShapeDtypeStruct((_N, _D_HID), jnp.float32),
            jax.ShapeDtypeStruct((_N, 1), jnp.float32),
        ],
    )(agg, deg_col, x, W1_l, b1.reshape(1, -1), W1_r)


def _tc2_body(p_ref, di_ref, h_ref, wl_ref, b_ref, wr_ref, o_ref):
    agg = p_ref[...] * di_ref[...]
    h = (jnp.dot(agg, wl_ref[...], preferred_element_type=jnp.float32,
                 precision=lax.Precision.HIGHEST)
         + b_ref[...]
         + jnp.dot(h_ref[...], wr_ref[...], preferred_element_type=jnp.float32,
                   precision=lax.Precision.HIGHEST))
    h = jax.nn.sigmoid(h)
    nrm = jnp.sqrt(jnp.sum(h * h, axis=1, keepdims=True))
    h = h / jnp.maximum(nrm, 1e-12)
    m = jnp.max(h, axis=1, keepdims=True)
    lse = m + jnp.log(jnp.sum(jnp.exp(h - m), axis=1, keepdims=True))
    o_ref[...] = h - lse


def _tc2(agg, deg_inv, h1, W2_l, b2, W2_r):
    grid = _N // _BLK
    return pl.pallas_call(
        _tc2_body,
        grid=(grid,),
        in_specs=[
            pl.BlockSpec((_BLK, _D_HID), lambda i: (i, 0)),
            pl.BlockSpec((_BLK, 1), lambda i: (i, 0)),
            pl.BlockSpec((_BLK, _D_HID), lambda i: (i, 0)),
            pl.BlockSpec((_D_HID, _D_OUT), lambda i: (0, 0)),
            pl.BlockSpec((1, _D_OUT), lambda i: (0, 0)),
            pl.BlockSpec((_D_HID, _D_OUT), lambda i: (0, 0)),
        ],
        out_specs=pl.BlockSpec((_BLK, _D_OUT), lambda i: (i, 0)),
        out_shape=jax.ShapeDtypeStruct((_N, _D_OUT), jnp.float32),
    )(agg, deg_inv, h1, W2_l, b2.reshape(1, -1), W2_r)


def _edge_tables(edge_index):
    srcf = edge_index[0].reshape(1, _NS, _CPW, _K)
    srcf = jnp.broadcast_to(srcf, (_NC, _NS, _CPW, _K))
    dstf = edge_index[1].reshape(1, _NS, _CPW, _K)
    cid = jnp.arange(_NC, dtype=jnp.int32)[:, None, None, None]
    sid = jnp.arange(_NS, dtype=jnp.int32)[None, :, None, None]
    ln = dstf - cid * _NH
    inc = (ln >= 0) & (ln < _NH)
    dstc = jnp.where(inc, ln, _NH)
    wid = cid * _NS + sid
    dr = jax.lax.shift_right_logical(dstc, 7) + sid * _DGR
    dc = jax.lax.bitwise_and(dstc, 127) * (_NC * _NS) + wid
    tbl = jnp.stack([srcf, dstc, dr, dc], axis=3)
    pad = jnp.zeros((_NC, _NS, _IDXS, 4, _K), jnp.int32)
    return jnp.concatenate([tbl, pad], axis=2)


def kernel(x, edge_index, W1_l, b1, W1_r, W2_l, b2, W2_r):
    tbl = _edge_tables(edge_index)
    ident = jnp.repeat(jnp.eye(128, dtype=jnp.float32), _NC * _NS, axis=0)
    zeros = jnp.zeros((_ACC_R, _D_IN), jnp.float32)

    parts1, deg = _sc_agg(True)(x, tbl, zeros, ident)
    agg1 = parts1[:, :_NH].reshape(_N, _D_IN)
    deg_col = (deg.reshape(_NC, _NS, _DGR * 128).sum(axis=1)[:, :_NH]
               .reshape(_N, 1))
    h1, deg_inv = _tc1(agg1, deg_col, x, W1_l, b1, W1_r)
    parts2 = _sc_agg(False)(h1, tbl, zeros)[0]
    agg2 = parts2[:, :_NH].reshape(_N, _D_HID)
    return _tc2(agg2, deg_inv, h1, W2_l, b2, W2_r)


# speedup vs baseline: 14.2006x; 1.5005x over previous
import functools

import jax
import jax.numpy as jnp
from jax import lax
from jax.experimental import pallas as pl
from jax.experimental.pallas import tpu as pltpu
from jax.experimental.pallas import tpu_sc as plsc

_N = 10000
_E = 320000
_D_IN = 128
_D_HID = 128
_D_OUT = 256

_NC = 2
_NS = 16
_NH = _N // _NC
_ACC_R = _NH + 8
_K = 40
_EPW = _E // _NS
_CPW = _EPW // _K
_NBUF = 2
_IDXS = 4
_RPT = 312
_TAIL = _ACC_R - _NS * _RPT

_DGG = 80


def _make_sc_agg(with_deg):
    mesh = plsc.VectorSubcoreMesh(
        core_axis_name="c", subcore_axis_name="s",
        num_cores=_NC, num_subcores=_NS)

    out_type = [jax.ShapeDtypeStruct((_NC, _ACC_R, _D_IN), jnp.float32)]
    scratch = [
        pltpu.VMEM((_IDXS, 4, _K), jnp.int32),
        pltpu.VMEM((_NBUF, _K, _D_IN), jnp.float32),
        pltpu.VMEM_SHARED((_ACC_R, _D_IN), jnp.float32),
        pltpu.SemaphoreType.DMA((_NBUF,)),
        pltpu.SemaphoreType.DMA((_IDXS,)),
    ]
    if with_deg:
        out_type.append(
            jax.ShapeDtypeStruct((_NC, _NS * _DGG, 128), jnp.float32))
        scratch += [
            pltpu.VMEM((_NBUF, _K, 128), jnp.float32),
            pltpu.VMEM_SHARED((_NS * _DGG, 128), jnp.float32),
            pltpu.SemaphoreType.DMA((_NBUF,)),
        ]

    @functools.partial(
        pl.kernel, out_type=tuple(out_type), mesh=mesh,
        scratch_types=scratch)
    def agg(*refs):
        if with_deg:
            (x_hbm, tbl_hbm, zero_hbm, ident_hbm,
             out_hbm, deg_hbm,
             idx_v, rows_v, acc, sems_g, sems_i,
             orow_v, deg_acc, sems_o) = refs
        else:
            (x_hbm, tbl_hbm, zero_hbm,
             out_hbm,
             idx_v, rows_v, acc, sems_g, sems_i) = refs
        c = lax.axis_index("c")
        s = lax.axis_index("s")

        pltpu.sync_copy(zero_hbm.at[pl.ds(s * _RPT, _RPT)],
                        acc.at[pl.ds(s * _RPT, _RPT)])

        @pl.when(s == _NS - 1)
        def _zero_tail():
            pltpu.sync_copy(zero_hbm.at[pl.ds(_NS * _RPT, _TAIL)],
                            acc.at[pl.ds(_NS * _RPT, _TAIL)])

        if with_deg:
            pltpu.sync_copy(zero_hbm.at[pl.ds(0, _DGG)],
                            deg_acc.at[pl.ds(s * _DGG, _DGG)])

        plsc.subcore_barrier()

        def i_copy(g, t):
            return pltpu.make_async_copy(
                tbl_hbm.at[c, s, g], idx_v.at[t], sems_i.at[t])

        def g_copy(g, b):
            return pltpu.make_async_copy(
                x_hbm.at[idx_v.at[lax.rem(g, _IDXS), 0]], rows_v.at[b],
                sems_g.at[b])

        def o_copy(ss, db):
            return pltpu.make_async_copy(
                ident_hbm.at[idx_v.at[lax.rem(2 * ss + c, _IDXS), 3]],
                orow_v.at[db], sems_o.at[db])

        def deg_step(ss):
            db = lax.rem(ss, _NBUF)
            o_copy(ss, db).wait()
            pltpu.sync_copy(
                orow_v.at[db],
                deg_acc.at[idx_v.at[lax.rem(2 * ss + c, _IDXS), 2]],
                add=True)

        def chunk_body(g, b, last):
            sl = lax.rem(g, _IDXS)
            g_copy(g, b).wait()
            pltpu.sync_copy(rows_v.at[b], acc.at[idx_v.at[sl, 1]], add=True)
            if not last:
                i_copy(g + _IDXS, sl).start()
                i_copy(g + _NBUF, lax.rem(g + _NBUF, _IDXS)).wait()
                g_copy(g + _NBUF, b).start()

        for t in range(_IDXS):
            i_copy(t, t).start()
        for b in range(_NBUF):
            i_copy(b, b).wait()
            g_copy(b, b).start()
        if with_deg:
            o_copy(0, 0).start()

        def super_step(ss, carry):
            if with_deg:
                deg_step(ss)
            chunk_body(ss * _NBUF, 0, last=False)
            chunk_body(ss * _NBUF + 1, 1, last=False)
            if with_deg:
                o_copy(ss + 1, lax.rem(ss + 1, _NBUF)).start()
            return carry

        _LAST_SS = _CPW // _NBUF - 1
        lax.fori_loop(0, _LAST_SS, super_step, 0)
        if with_deg:
            deg_step(_LAST_SS)
        for b in range(_NBUF):
            chunk_body(_CPW - _NBUF + b, b, last=True)
        for b in range(_NBUF):
            g = _CPW + b
            i_copy(g, lax.rem(g, _IDXS)).wait()

        plsc.subcore_barrier()
        pltpu.sync_copy(acc.at[pl.ds(s * _RPT, _RPT)],
                        out_hbm.at[c, pl.ds(s * _RPT, _RPT)])

        @pl.when(s == _NS - 1)
        def _write_tail():
            pltpu.sync_copy(acc.at[pl.ds(_NS * _RPT, _TAIL)],
                            out_hbm.at[c, pl.ds(_NS * _RPT, _TAIL)])

        if with_deg:
            pltpu.sync_copy(deg_acc.at[pl.ds(s * _DGG, _DGG)],
                            deg_hbm.at[c, pl.ds(s * _DGG, _DGG)])

    return agg


@functools.cache
def _sc_agg(with_deg):
    return _make_sc_agg(with_deg)


_BLK = 2000


def _tc1_body(p_ref, deg_ref, x_ref, wl_ref, b_ref, wr_ref, h_ref, di_ref):
    deg = jnp.maximum(deg_ref[...], 1.0)
    deg_inv = 1.0 / deg
    agg = p_ref[...] * deg_inv
    h = (jnp.dot(agg, wl_ref[...], preferred_element_type=jnp.float32,
                 precision=lax.Precision.HIGHEST)
         + b_ref[...]
         + jnp.dot(x_ref[...], wr_ref[...], preferred_element_type=jnp.float32,
                   precision=lax.Precision.HIGHEST))
    h = jax.nn.sigmoid(h)
    nrm = jnp.sqrt(jnp.sum(h * h, axis=1, keepdims=True))
    h_ref[...] = h / jnp.maximum(nrm, 1e-12)
    di_ref[...] = deg_inv


def _tc1(agg, deg_col, x, W1_l, b1, W1_r):
    grid = _N // _BLK
    return pl.pallas_call(
        _tc1_body,
        grid=(grid,),
        in_specs=[
            pl.BlockSpec((_BLK, _D_IN), lambda i: (i, 0)),
            pl.BlockSpec((_BLK, 1), lambda i: (i, 0)),
            pl.BlockSpec((_BLK, _D_IN), lambda i: (i, 0)),
            pl.BlockSpec((_D_IN, _D_HID), lambda i: (0, 0)),
            pl.BlockSpec((1, _D_HID), lambda i: (0, 0)),
            pl.BlockSpec((_D_IN, _D_HID), lambda i: (0, 0)),
        ],
        out_specs=[
            pl.BlockSpec((_BLK, _D_HID), lambda i: (i, 0)),
            pl.BlockSpec((_BLK, 1), lambda i: (i, 0)),
        ],
        out_shape=[
            jax.ShapeDtypeStruct((_N, _D_HID), jnp.float32),
            jax.ShapeDtypeStruct((_N, 1), jnp.float32),
        ],
    )(agg, deg_col, x, W1_l, b1.reshape(1, -1), W1_r)


def _tc2_body(p_ref, di_ref, h_ref, wl_ref, b_ref, wr_ref, o_ref):
    agg = p_ref[...] * di_ref[...]
    h = (jnp.dot(agg, wl_ref[...], preferred_element_type=jnp.float32,
                 precision=lax.Precision.HIGHEST)
         + b_ref[...]
         + jnp.dot(h_ref[...], wr_ref[...], preferred_element_type=jnp.float32,
                   precision=lax.Precision.HIGHEST))
    h = jax.nn.sigmoid(h)
    nrm = jnp.sqrt(jnp.sum(h * h, axis=1, keepdims=True))
    h = h / jnp.maximum(nrm, 1e-12)
    m = jnp.max(h, axis=1, keepdims=True)
    lse = m + jnp.log(jnp.sum(jnp.exp(h - m), axis=1, keepdims=True))
    o_ref[...] = h - lse


def _tc2(agg, deg_inv, h1, W2_l, b2, W2_r):
    grid = _N // _BLK
    return pl.pallas_call(
        _tc2_body,
        grid=(grid,),
        in_specs=[
            pl.BlockSpec((_BLK, _D_HID), lambda i: (i, 0)),
            pl.BlockSpec((_BLK, 1), lambda i: (i, 0)),
            pl.BlockSpec((_BLK, _D_HID), lambda i: (i, 0)),
            pl.BlockSpec((_D_HID, _D_OUT), lambda i: (0, 0)),
            pl.BlockSpec((1, _D_OUT), lambda i: (0, 0)),
            pl.BlockSpec((_D_HID, _D_OUT), lambda i: (0, 0)),
        ],
        out_specs=pl.BlockSpec((_BLK, _D_OUT), lambda i: (i, 0)),
        out_shape=jax.ShapeDtypeStruct((_N, _D_OUT), jnp.float32),
    )(agg, deg_inv, h1, W2_l, b2.reshape(1, -1), W2_r)


def _edge_tables(edge_index):
    srcf = edge_index[0].reshape(1, _NS, _CPW, _K)
    srcf = jnp.broadcast_to(srcf, (_NC, _NS, _CPW, _K))
    dstf = edge_index[1].reshape(1, _NS, _CPW, _K)
    cid = jnp.arange(_NC, dtype=jnp.int32)[:, None, None, None]
    sid = jnp.arange(_NS, dtype=jnp.int32)[None, :, None, None]
    ln = dstf - cid * _NH
    inc = (ln >= 0) & (ln < _NH)
    dstc = jnp.where(inc, ln, _NH)
    wid = cid * _NS + sid
    dr = jax.lax.shift_right_logical(dstf, 7) + sid * _DGG
    dc = jax.lax.bitwise_and(dstf, 127) * (_NC * _NS) + wid
    dr = jnp.broadcast_to(dr, (_NC, _NS, _CPW, _K))
    dc = jnp.broadcast_to(dc, (_NC, _NS, _CPW, _K))
    tbl = jnp.stack([srcf, dstc, dr, dc], axis=3)
    pad = jnp.zeros((_NC, _NS, _IDXS, 4, _K), jnp.int32)
    return jnp.concatenate([tbl, pad], axis=2)


def kernel(x, edge_index, W1_l, b1, W1_r, W2_l, b2, W2_r):
    tbl = _edge_tables(edge_index)
    ident = jnp.repeat(jnp.eye(128, dtype=jnp.float32), _NC * _NS, axis=0)
    zeros = jnp.zeros((_ACC_R, _D_IN), jnp.float32)

    parts1, deg = _sc_agg(True)(x, tbl, zeros, ident)
    agg1 = parts1[:, :_NH].reshape(_N, _D_IN)
    deg_col = (deg.reshape(_NC, _NS, _DGG * 128)[:, :, :_N]
               .sum(axis=(0, 1)).reshape(_N, 1))
    h1, deg_inv = _tc1(agg1, deg_col, x, W1_l, b1, W1_r)
    parts2 = _sc_agg(False)(h1, tbl, zeros)[0]
    agg2 = parts2[:, :_NH].reshape(_N, _D_HID)
    return _tc2(agg2, deg_inv, h1, W2_l, b2, W2_r)


# speedup vs baseline: 14.3702x; 1.0119x over previous
import functools

import jax
import jax.numpy as jnp
from jax import lax
from jax.experimental import pallas as pl
from jax.experimental.pallas import tpu as pltpu
from jax.experimental.pallas import tpu_sc as plsc

_N = 10000
_E = 320000
_D_IN = 128
_D_HID = 128
_D_OUT = 256

_NC = 2
_NS = 16
_NH = _N // _NC
_NJ = 64
_ACC_R = _NH + _NJ
_K = 40
_EPW = _E // _NS
_CPW = _EPW // _K
_NBUF = 2
_IDXS = 4
_RPT = 312
_TAIL = _ACC_R - _NS * _RPT

_DGG = 80


def _make_sc_agg(with_deg):
    mesh = plsc.VectorSubcoreMesh(
        core_axis_name="c", subcore_axis_name="s",
        num_cores=_NC, num_subcores=_NS)

    out_type = [jax.ShapeDtypeStruct((_NC, _ACC_R, _D_IN), jnp.float32)]
    scratch = [
        pltpu.VMEM((_IDXS, 4, _K), jnp.int32),
        pltpu.VMEM((_NBUF, _K, _D_IN), jnp.float32),
        pltpu.VMEM_SHARED((_ACC_R, _D_IN), jnp.float32),
        pltpu.SemaphoreType.DMA((_NBUF,)),
        pltpu.SemaphoreType.DMA((_IDXS,)),
    ]
    if with_deg:
        out_type.append(
            jax.ShapeDtypeStruct((_NC, _NS * _DGG, 128), jnp.float32))
        scratch += [
            pltpu.VMEM((_NBUF, _K, 128), jnp.float32),
            pltpu.VMEM_SHARED((_NS * _DGG, 128), jnp.float32),
            pltpu.SemaphoreType.DMA((_NBUF,)),
        ]

    @functools.partial(
        pl.kernel, out_type=tuple(out_type), mesh=mesh,
        scratch_types=scratch)
    def agg(*refs):
        if with_deg:
            (x_hbm, tbl_hbm, zero_hbm, ident_hbm,
             out_hbm, deg_hbm,
             idx_v, rows_v, acc, sems_g, sems_i,
             orow_v, deg_acc, sems_o) = refs
        else:
            (x_hbm, tbl_hbm, zero_hbm,
             out_hbm,
             idx_v, rows_v, acc, sems_g, sems_i) = refs
        c = lax.axis_index("c")
        s = lax.axis_index("s")

        pltpu.sync_copy(zero_hbm.at[pl.ds(s * _RPT, _RPT)],
                        acc.at[pl.ds(s * _RPT, _RPT)])

        @pl.when(s == _NS - 1)
        def _zero_tail():
            pltpu.sync_copy(zero_hbm.at[pl.ds(_NS * _RPT, _TAIL)],
                            acc.at[pl.ds(_NS * _RPT, _TAIL)])

        if with_deg:
            pltpu.sync_copy(zero_hbm.at[pl.ds(0, _DGG)],
                            deg_acc.at[pl.ds(s * _DGG, _DGG)])

        plsc.subcore_barrier()

        def i_copy(g, t):
            return pltpu.make_async_copy(
                tbl_hbm.at[c, s, g], idx_v.at[t], sems_i.at[t])

        def g_copy(g, b):
            return pltpu.make_async_copy(
                x_hbm.at[idx_v.at[lax.rem(g, _IDXS), 0]], rows_v.at[b],
                sems_g.at[b])

        def o_copy(ss, db):
            return pltpu.make_async_copy(
                ident_hbm.at[idx_v.at[lax.rem(2 * ss + c, _IDXS), 3]],
                orow_v.at[db], sems_o.at[db])

        def deg_step(ss):
            db = lax.rem(ss, _NBUF)
            o_copy(ss, db).wait()
            pltpu.sync_copy(
                orow_v.at[db],
                deg_acc.at[idx_v.at[lax.rem(2 * ss + c, _IDXS), 2]],
                add=True)

        def chunk_body(g, b, last):
            sl = lax.rem(g, _IDXS)
            g_copy(g, b).wait()
            pltpu.sync_copy(rows_v.at[b], acc.at[idx_v.at[sl, 1]], add=True)
            if not last:
                i_copy(g + _IDXS, sl).start()
                i_copy(g + _NBUF, lax.rem(g + _NBUF, _IDXS)).wait()
                g_copy(g + _NBUF, b).start()

        for t in range(_IDXS):
            i_copy(t, t).start()
        for b in range(_NBUF):
            i_copy(b, b).wait()
            g_copy(b, b).start()
        if with_deg:
            o_copy(0, 0).start()

        def super_step(ss, carry):
            if with_deg:
                deg_step(ss)
            chunk_body(ss * _NBUF, 0, last=False)
            chunk_body(ss * _NBUF + 1, 1, last=False)
            if with_deg:
                o_copy(ss + 1, lax.rem(ss + 1, _NBUF)).start()
            return carry

        _LAST_SS = _CPW // _NBUF - 1
        lax.fori_loop(0, _LAST_SS, super_step, 0)
        if with_deg:
            deg_step(_LAST_SS)
        for b in range(_NBUF):
            chunk_body(_CPW - _NBUF + b, b, last=True)
        for b in range(_NBUF):
            g = _CPW + b
            i_copy(g, lax.rem(g, _IDXS)).wait()

        plsc.subcore_barrier()
        pltpu.sync_copy(acc.at[pl.ds(s * _RPT, _RPT)],
                        out_hbm.at[c, pl.ds(s * _RPT, _RPT)])

        @pl.when(s == _NS - 1)
        def _write_tail():
            pltpu.sync_copy(acc.at[pl.ds(_NS * _RPT, _TAIL)],
                            out_hbm.at[c, pl.ds(_NS * _RPT, _TAIL)])

        if with_deg:
            pltpu.sync_copy(deg_acc.at[pl.ds(s * _DGG, _DGG)],
                            deg_hbm.at[c, pl.ds(s * _DGG, _DGG)])

    return agg


@functools.cache
def _sc_agg(with_deg):
    return _make_sc_agg(with_deg)


_BLK = 2000


def _tc1_body(p_ref, deg_ref, x_ref, wl_ref, b_ref, wr_ref, h_ref, di_ref):
    deg = jnp.maximum(deg_ref[...], 1.0)
    deg_inv = 1.0 / deg
    agg = p_ref[...] * deg_inv
    h = (jnp.dot(agg, wl_ref[...], preferred_element_type=jnp.float32,
                 precision=lax.Precision.HIGHEST)
         + b_ref[...]
         + jnp.dot(x_ref[...], wr_ref[...], preferred_element_type=jnp.float32,
                   precision=lax.Precision.HIGHEST))
    h = jax.nn.sigmoid(h)
    nrm = jnp.sqrt(jnp.sum(h * h, axis=1, keepdims=True))
    h_ref[...] = h / jnp.maximum(nrm, 1e-12)
    di_ref[...] = deg_inv


def _tc1(agg, deg_col, x, W1_l, b1, W1_r):
    grid = _N // _BLK
    return pl.pallas_call(
        _tc1_body,
        grid=(grid,),
        in_specs=[
            pl.BlockSpec((_BLK, _D_IN), lambda i: (i, 0)),
            pl.BlockSpec((_BLK, 1), lambda i: (i, 0)),
            pl.BlockSpec((_BLK, _D_IN), lambda i: (i, 0)),
            pl.BlockSpec((_D_IN, _D_HID), lambda i: (0, 0)),
            pl.BlockSpec((1, _D_HID), lambda i: (0, 0)),
            pl.BlockSpec((_D_IN, _D_HID), lambda i: (0, 0)),
        ],
        out_specs=[
            pl.BlockSpec((_BLK, _D_HID), lambda i: (i, 0)),
            pl.BlockSpec((_BLK, 1), lambda i: (i, 0)),
        ],
        out_shape=[
            jax.ShapeDtypeStruct((_N, _D_HID), jnp.float32),
            jax.ShapeDtypeStruct((_N, 1), jnp.float32),
        ],
    )(agg, deg_col, x, W1_l, b1.reshape(1, -1), W1_r)


def _tc2_body(p_ref, di_ref, h_ref, wl_ref, b_ref, wr_ref, o_ref):
    agg = p_ref[...] * di_ref[...]
    h = (jnp.dot(agg, wl_ref[...], preferred_element_type=jnp.float32,
                 precision=lax.Precision.HIGHEST)
         + b_ref[...]
         + jnp.dot(h_ref[...], wr_ref[...], preferred_element_type=jnp.float32,
                   precision=lax.Precision.HIGHEST))
    h = jax.nn.sigmoid(h)
    nrm = jnp.sqrt(jnp.sum(h * h, axis=1, keepdims=True))
    h = h / jnp.maximum(nrm, 1e-12)
    m = jnp.max(h, axis=1, keepdims=True)
    lse = m + jnp.log(jnp.sum(jnp.exp(h - m), axis=1, keepdims=True))
    o_ref[...] = h - lse


def _tc2(agg, deg_inv, h1, W2_l, b2, W2_r):
    grid = _N // _BLK
    return pl.pallas_call(
        _tc2_body,
        grid=(grid,),
        in_specs=[
            pl.BlockSpec((_BLK, _D_HID), lambda i: (i, 0)),
            pl.BlockSpec((_BLK, 1), lambda i: (i, 0)),
            pl.BlockSpec((_BLK, _D_HID), lambda i: (i, 0)),
            pl.BlockSpec((_D_HID, _D_OUT), lambda i: (0, 0)),
            pl.BlockSpec((1, _D_OUT), lambda i: (0, 0)),
            pl.BlockSpec((_D_HID, _D_OUT), lambda i: (0, 0)),
        ],
        out_specs=pl.BlockSpec((_BLK, _D_OUT), lambda i: (i, 0)),
        out_shape=jax.ShapeDtypeStruct((_N, _D_OUT), jnp.float32),
    )(agg, deg_inv, h1, W2_l, b2.reshape(1, -1), W2_r)


def _edge_tables(edge_index):
    srcf = edge_index[0].reshape(1, _NS, _CPW, _K)
    srcf = jnp.broadcast_to(srcf, (_NC, _NS, _CPW, _K))
    dstf = edge_index[1].reshape(1, _NS, _CPW, _K)
    cid = jnp.arange(_NC, dtype=jnp.int32)[:, None, None, None]
    sid = jnp.arange(_NS, dtype=jnp.int32)[None, :, None, None]
    ln = dstf - cid * _NH
    inc = (ln >= 0) & (ln < _NH)
    spread = jax.lax.bitwise_and(
        jnp.arange(_E, dtype=jnp.int32).reshape(1, _NS, _CPW, _K), _NJ - 1)
    dstc = jnp.where(inc, ln, _NH + spread)
    wid = cid * _NS + sid
    dr = jax.lax.shift_right_logical(dstf, 7) + sid * _DGG
    dc = jax.lax.bitwise_and(dstf, 127) * (_NC * _NS) + wid
    dr = jnp.broadcast_to(dr, (_NC, _NS, _CPW, _K))
    dc = jnp.broadcast_to(dc, (_NC, _NS, _CPW, _K))
    tbl = jnp.stack([srcf, dstc, dr, dc], axis=3)
    pad = jnp.zeros((_NC, _NS, _IDXS, 4, _K), jnp.int32)
    return jnp.concatenate([tbl, pad], axis=2)


def kernel(x, edge_index, W1_l, b1, W1_r, W2_l, b2, W2_r):
    tbl = _edge_tables(edge_index)
    ident = jnp.repeat(jnp.eye(128, dtype=jnp.float32), _NC * _NS, axis=0)
    zeros = jnp.zeros((_ACC_R, _D_IN), jnp.float32)

    parts1, deg = _sc_agg(True)(x, tbl, zeros, ident)
    agg1 = parts1[:, :_NH].reshape(_N, _D_IN)
    deg_col = (deg.reshape(_NC, _NS, _DGG * 128)[:, :, :_N]
               .sum(axis=(0, 1)).reshape(_N, 1))
    h1, deg_inv = _tc1(agg1, deg_col, x, W1_l, b1, W1_r)
    parts2 = _sc_agg(False)(h1, tbl, zeros)[0]
    agg2 = parts2[:, :_NH].reshape(_N, _D_HID)
    return _tc2(agg2, deg_inv, h1, W2_l, b2, W2_r)


# speedup vs baseline: 17.8688x; 1.2435x over previous
import functools

import jax
import jax.numpy as jnp
from jax import lax
from jax.experimental import pallas as pl
from jax.experimental.pallas import tpu as pltpu
from jax.experimental.pallas import tpu_sc as plsc

_N = 10000
_E = 320000
_D_IN = 128
_D_HID = 128
_D_OUT = 256

_NC = 2
_NS = 16
_NH = _N // _NC
_NJ = 64
_ACC_R = _NH + _NJ
_K = 80
_EPW = _E // _NS
_CPW = _EPW // _K
_NBUF = 2
_IDXS = 4
_RPT = 312
_TAIL = _ACC_R - _NS * _RPT

_DGG = 80


def _make_sc_agg(with_deg):
    mesh = plsc.VectorSubcoreMesh(
        core_axis_name="c", subcore_axis_name="s",
        num_cores=_NC, num_subcores=_NS)

    out_type = [jax.ShapeDtypeStruct((_NC, _ACC_R, _D_IN), jnp.float32)]
    scratch = [
        pltpu.VMEM((_IDXS, 4, _K), jnp.int32),
        pltpu.VMEM((_NBUF, _K, _D_IN), jnp.float32),
        pltpu.VMEM_SHARED((_ACC_R, _D_IN), jnp.float32),
        pltpu.SemaphoreType.DMA((_NBUF,)),
        pltpu.SemaphoreType.DMA((_IDXS,)),
    ]
    if with_deg:
        out_type.append(
            jax.ShapeDtypeStruct((_NC, _NS * _DGG, 128), jnp.float32))
        scratch += [
            pltpu.VMEM((_NBUF, _K, 128), jnp.float32),
            pltpu.VMEM_SHARED((_NS * _DGG, 128), jnp.float32),
            pltpu.SemaphoreType.DMA((_NBUF,)),
        ]

    @functools.partial(
        pl.kernel, out_type=tuple(out_type), mesh=mesh,
        scratch_types=scratch)
    def agg(*refs):
        if with_deg:
            (x_hbm, tbl_hbm, zero_hbm, ident_hbm,
             out_hbm, deg_hbm,
             idx_v, rows_v, acc, sems_g, sems_i,
             orow_v, deg_acc, sems_o) = refs
        else:
            (x_hbm, tbl_hbm, zero_hbm,
             out_hbm,
             idx_v, rows_v, acc, sems_g, sems_i) = refs
        c = lax.axis_index("c")
        s = lax.axis_index("s")

        pltpu.sync_copy(zero_hbm.at[pl.ds(s * _RPT, _RPT)],
                        acc.at[pl.ds(s * _RPT, _RPT)])

        @pl.when(s == _NS - 1)
        def _zero_tail():
            pltpu.sync_copy(zero_hbm.at[pl.ds(_NS * _RPT, _TAIL)],
                            acc.at[pl.ds(_NS * _RPT, _TAIL)])

        if with_deg:
            pltpu.sync_copy(zero_hbm.at[pl.ds(0, _DGG)],
                            deg_acc.at[pl.ds(s * _DGG, _DGG)])

        plsc.subcore_barrier()

        def i_copy(g, t):
            return pltpu.make_async_copy(
                tbl_hbm.at[c, s, g], idx_v.at[t], sems_i.at[t])

        def g_copy(g, b):
            return pltpu.make_async_copy(
                x_hbm.at[idx_v.at[lax.rem(g, _IDXS), 0]], rows_v.at[b],
                sems_g.at[b])

        def o_copy(ss, db):
            return pltpu.make_async_copy(
                ident_hbm.at[idx_v.at[lax.rem(2 * ss + c, _IDXS), 3]],
                orow_v.at[db], sems_o.at[db])

        def deg_step(ss):
            db = lax.rem(ss, _NBUF)
            o_copy(ss, db).wait()
            pltpu.sync_copy(
                orow_v.at[db],
                deg_acc.at[idx_v.at[lax.rem(2 * ss + c, _IDXS), 2]],
                add=True)

        def chunk_body(g, b, last):
            sl = lax.rem(g, _IDXS)
            g_copy(g, b).wait()
            pltpu.sync_copy(rows_v.at[b], acc.at[idx_v.at[sl, 1]], add=True)
            if not last:
                i_copy(g + _IDXS, sl).start()
                i_copy(g + _NBUF, lax.rem(g + _NBUF, _IDXS)).wait()
                g_copy(g + _NBUF, b).start()

        for t in range(_IDXS):
            i_copy(t, t).start()
        for b in range(_NBUF):
            i_copy(b, b).wait()
            g_copy(b, b).start()
        if with_deg:
            o_copy(0, 0).start()

        def super_step(ss, carry):
            if with_deg:
                deg_step(ss)
            chunk_body(ss * _NBUF, 0, last=False)
            chunk_body(ss * _NBUF + 1, 1, last=False)
            if with_deg:
                o_copy(ss + 1, lax.rem(ss + 1, _NBUF)).start()
            return carry

        _LAST_SS = _CPW // _NBUF - 1
        lax.fori_loop(0, _LAST_SS, super_step, 0)
        if with_deg:
            deg_step(_LAST_SS)
        for b in range(_NBUF):
            chunk_body(_CPW - _NBUF + b, b, last=True)
        for b in range(_NBUF):
            g = _CPW + b
            i_copy(g, lax.rem(g, _IDXS)).wait()

        plsc.subcore_barrier()
        pltpu.sync_copy(acc.at[pl.ds(s * _RPT, _RPT)],
                        out_hbm.at[c, pl.ds(s * _RPT, _RPT)])

        @pl.when(s == _NS - 1)
        def _write_tail():
            pltpu.sync_copy(acc.at[pl.ds(_NS * _RPT, _TAIL)],
                            out_hbm.at[c, pl.ds(_NS * _RPT, _TAIL)])

        if with_deg:
            pltpu.sync_copy(deg_acc.at[pl.ds(s * _DGG, _DGG)],
                            deg_hbm.at[c, pl.ds(s * _DGG, _DGG)])

    return agg


@functools.cache
def _sc_agg(with_deg):
    return _make_sc_agg(with_deg)


_BLK = 2000


def _tc1_body(p_ref, deg_ref, x_ref, wl_ref, b_ref, wr_ref, h_ref, di_ref):
    deg = jnp.maximum(deg_ref[...], 1.0)
    deg_inv = 1.0 / deg
    agg = p_ref[...] * deg_inv
    h = (jnp.dot(agg, wl_ref[...], preferred_element_type=jnp.float32,
                 precision=lax.Precision.HIGHEST)
         + b_ref[...]
         + jnp.dot(x_ref[...], wr_ref[...], preferred_element_type=jnp.float32,
                   precision=lax.Precision.HIGHEST))
    h = jax.nn.sigmoid(h)
    nrm = jnp.sqrt(jnp.sum(h * h, axis=1, keepdims=True))
    h_ref[...] = h / jnp.maximum(nrm, 1e-12)
    di_ref[...] = deg_inv


def _tc1(agg, deg_col, x, W1_l, b1, W1_r):
    grid = _N // _BLK
    return pl.pallas_call(
        _tc1_body,
        grid=(grid,),
        in_specs=[
            pl.BlockSpec((_BLK, _D_IN), lambda i: (i, 0)),
            pl.BlockSpec((_BLK, 1), lambda i: (i, 0)),
            pl.BlockSpec((_BLK, _D_IN), lambda i: (i, 0)),
            pl.BlockSpec((_D_IN, _D_HID), lambda i: (0, 0)),
            pl.BlockSpec((1, _D_HID), lambda i: (0, 0)),
            pl.BlockSpec((_D_IN, _D_HID), lambda i: (0, 0)),
        ],
        out_specs=[
            pl.BlockSpec((_BLK, _D_HID), lambda i: (i, 0)),
            pl.BlockSpec((_BLK, 1), lambda i: (i, 0)),
        ],
        out_shape=[
            jax.ShapeDtypeStruct((_N, _D_HID), jnp.float32),
            jax.ShapeDtypeStruct((_N, 1), jnp.float32),
        ],
    )(agg, deg_col, x, W1_l, b1.reshape(1, -1), W1_r)


def _tc2_body(p_ref, di_ref, h_ref, wl_ref, b_ref, wr_ref, o_ref):
    agg = p_ref[...] * di_ref[...]
    h = (jnp.dot(agg, wl_ref[...], preferred_element_type=jnp.float32,
                 precision=lax.Precision.HIGHEST)
         + b_ref[...]
         + jnp.dot(h_ref[...], wr_ref[...], preferred_element_type=jnp.float32,
                   precision=lax.Precision.HIGHEST))
    h = jax.nn.sigmoid(h)
    nrm = jnp.sqrt(jnp.sum(h * h, axis=1, keepdims=True))
    h = h / jnp.maximum(nrm, 1e-12)
    m = jnp.max(h, axis=1, keepdims=True)
    lse = m + jnp.log(jnp.sum(jnp.exp(h - m), axis=1, keepdims=True))
    o_ref[...] = h - lse


def _tc2(agg, deg_inv, h1, W2_l, b2, W2_r):
    grid = _N // _BLK
    return pl.pallas_call(
        _tc2_body,
        grid=(grid,),
        in_specs=[
            pl.BlockSpec((_BLK, _D_HID), lambda i: (i, 0)),
            pl.BlockSpec((_BLK, 1), lambda i: (i, 0)),
            pl.BlockSpec((_BLK, _D_HID), lambda i: (i, 0)),
            pl.BlockSpec((_D_HID, _D_OUT), lambda i: (0, 0)),
            pl.BlockSpec((1, _D_OUT), lambda i: (0, 0)),
            pl.BlockSpec((_D_HID, _D_OUT), lambda i: (0, 0)),
        ],
        out_specs=pl.BlockSpec((_BLK, _D_OUT), lambda i: (i, 0)),
        out_shape=jax.ShapeDtypeStruct((_N, _D_OUT), jnp.float32),
    )(agg, deg_inv, h1, W2_l, b2.reshape(1, -1), W2_r)


def _edge_tables(edge_index):
    srcf = edge_index[0].reshape(1, _NS, _CPW, _K)
    srcf = jnp.broadcast_to(srcf, (_NC, _NS, _CPW, _K))
    dstf = edge_index[1].reshape(1, _NS, _CPW, _K)
    cid = jnp.arange(_NC, dtype=jnp.int32)[:, None, None, None]
    sid = jnp.arange(_NS, dtype=jnp.int32)[None, :, None, None]
    ln = dstf - cid * _NH
    inc = (ln >= 0) & (ln < _NH)
    spread = jax.lax.bitwise_and(
        jnp.arange(_E, dtype=jnp.int32).reshape(1, _NS, _CPW, _K), _NJ - 1)
    dstc = jnp.where(inc, ln, _NH + spread)
    wid = cid * _NS + sid
    dr = jax.lax.shift_right_logical(dstf, 7) + sid * _DGG
    dc = jax.lax.bitwise_and(dstf, 127) * (_NC * _NS) + wid
    dr = jnp.broadcast_to(dr, (_NC, _NS, _CPW, _K))
    dc = jnp.broadcast_to(dc, (_NC, _NS, _CPW, _K))
    tbl = jnp.stack([srcf, dstc, dr, dc], axis=3)
    pad = jnp.zeros((_NC, _NS, _IDXS, 4, _K), jnp.int32)
    return jnp.concatenate([tbl, pad], axis=2)


def kernel(x, edge_index, W1_l, b1, W1_r, W2_l, b2, W2_r):
    tbl = _edge_tables(edge_index)
    ident = jnp.repeat(jnp.eye(128, dtype=jnp.float32), _NC * _NS, axis=0)
    zeros = jnp.zeros((_ACC_R, _D_IN), jnp.float32)

    parts1, deg = _sc_agg(True)(x, tbl, zeros, ident)
    agg1 = parts1[:, :_NH].reshape(_N, _D_IN)
    deg_col = (deg.reshape(_NC, _NS, _DGG * 128)[:, :, :_N]
               .sum(axis=(0, 1)).reshape(_N, 1))
    h1, deg_inv = _tc1(agg1, deg_col, x, W1_l, b1, W1_r)
    parts2 = _sc_agg(False)(h1, tbl, zeros)[0]
    agg2 = parts2[:, :_NH].reshape(_N, _D_HID)
    return _tc2(agg2, deg_inv, h1, W2_l, b2, W2_r)
